# baseline (speedup 1.0000x reference)
import functools

import jax
import jax.numpy as jnp
from jax import lax
from jax.experimental import pallas as pl
from jax.experimental.pallas import tpu as pltpu
from jax.experimental.pallas import tpu_sc as plsc

D = 64
NC, NS = 2, 16
NW = NC * NS
G = 128
NBUF = 4
LA = 2
CT = D // 8


def _make_kernel(BATCH, SEQ):
    assert BATCH == NW * G and (SEQ - NBUF) % NBUF == 0
    n_outer = (SEQ - NBUF) // NBUF
    mesh = plsc.VectorSubcoreMesh(core_axis_name="c", subcore_axis_name="s")

    @functools.partial(
        pl.kernel,
        mesh=mesh,
        out_type=jax.ShapeDtypeStruct((SEQ, CT, NW, 8 * G), jnp.float32),
        scratch_types=[
            pltpu.VMEM((SEQ, G), jnp.int32),
            pltpu.VMEM((NBUF * G, D), jnp.float32),
            pltpu.VMEM((NBUF * CT, 8 * G), jnp.float32),
        ]
        + [pltpu.SemaphoreType.DMA] * (2 * NBUF),
        compiler_params=pltpu.CompilerParams(
            use_tc_tiling_on_sc=False, needs_layout_passes=False
        ),
    )
    def k(table, idx_hbm, out_hbm, idx_all, rows, tiles, *sems):
        gsem, osem = sems[:NBUF], sems[NBUF:]
        wid = lax.axis_index("s") * NC + lax.axis_index("c")
        pltpu.sync_copy(idx_hbm.at[:, pl.ds(wid * G, G)], idx_all)
        iota = lax.iota(jnp.int32, 16)

        def fire(s, b):
            pltpu.async_copy(
                table.at[idx_all.at[s]],
                rows.at[pl.ds(b * G, G)],
                gsem[b],
            )

        def drain(b):
            pltpu.make_async_copy(
                table.at[pl.ds(0, G)], rows.at[pl.ds(b * G, G)], gsem[b]
            ).wait()

        def transpose(b):
            ridx = [iota + (b * G + 16 * k) for k in range(8)]

            def ct_body(ct, carry):
                for cm in range(8):
                    cidx = jnp.full((16,), 8 * ct + cm, jnp.int32)
                    for k in range(8):
                        vals = plsc.load_gather(rows, [ridx[k], cidx])
                        tiles[b * CT + ct, pl.ds(cm * G + 16 * k, 16)] = vals
                return carry

            lax.fori_loop(0, CT, ct_body, 0)

        def write(s, b):
            pltpu.async_copy(
                tiles.at[pl.ds(b * CT, CT)],
                out_hbm.at[s, :, wid, :],
                osem[b],
            )

        def wait_write(b):
            pltpu.make_async_copy(
                tiles.at[pl.ds(b * CT, CT)],
                out_hbm.at[0, :, 0, :],
                osem[b],
            ).wait()

        def retire(s, b):
            drain(b)
            transpose(b)
            write(s, b)

        fire(0, 0)
        fire(1, 1)
        fire(2, 2)
        retire(0, 0)
        fire(3, 3)
        retire(1, 1)

        def body(g, carry):
            s0 = NBUF + g * NBUF
            for b in range(NBUF):
                s = s0 + b
                wait_write(b)
                fire(s, b)
                retire(s - LA, (b + NBUF - LA) % NBUF)
            return carry

        lax.fori_loop(0, n_outer, body, 0)

        for s in range(SEQ - LA, SEQ):
            retire(s, s % NBUF)
        for b in range(NBUF):
            wait_write(b)

    return k


def kernel(indices, weight):
    BATCH, SEQ = indices.shape
    idx_t = jnp.transpose(indices.astype(jnp.int32))
    out4 = _make_kernel(BATCH, SEQ)(weight, idx_t)
    out5 = jnp.reshape(out4, (SEQ, CT, NW, 8, G))
    out = jnp.transpose(out5, (2, 4, 0, 1, 3))
    return jnp.reshape(out, (BATCH, SEQ, D))


# speedup vs baseline: 1.4754x; 1.4754x over previous
import functools

import jax
import jax.numpy as jnp
from jax import lax
from jax.experimental import pallas as pl
from jax.experimental.pallas import tpu as pltpu
from jax.experimental.pallas import tpu_sc as plsc

D = 64
NC, NS = 2, 16
NW = NC * NS
G = 128
NBUF = 4
LA = 2
CT = D // 8


def _make_kernel(BATCH, SEQ):
    assert BATCH == NW * G and (SEQ - NBUF) % NBUF == 0
    n_outer = (SEQ - NBUF) // NBUF
    mesh = plsc.VectorSubcoreMesh(core_axis_name="c", subcore_axis_name="s")

    @functools.partial(
        pl.kernel,
        mesh=mesh,
        out_type=jax.ShapeDtypeStruct((SEQ, CT, NW, 8 * G), jnp.float32),
        scratch_types=[
            pltpu.VMEM((SEQ, G), jnp.int32),
            pltpu.VMEM((NBUF * G, D), jnp.float32),
            pltpu.VMEM((NBUF * CT, 8 * G), jnp.float32),
        ]
        + [pltpu.SemaphoreType.DMA] * (2 * NBUF),
        compiler_params=pltpu.CompilerParams(
            use_tc_tiling_on_sc=False, needs_layout_passes=False
        ),
    )
    def k(table, idx_hbm, out_hbm, idx_all, rows, tiles, *sems):
        gsem, osem = sems[:NBUF], sems[NBUF:]
        wid = lax.axis_index("s") * NC + lax.axis_index("c")
        pltpu.sync_copy(idx_hbm.at[:, pl.ds(wid * G, G)], idx_all)
        iota = lax.iota(jnp.int32, 16)

        def fire(s, b):
            pltpu.async_copy(
                table.at[idx_all.at[s]],
                rows.at[pl.ds(b * G, G)],
                gsem[b],
            )

        def drain(b):
            pltpu.make_async_copy(
                table.at[pl.ds(0, G)], rows.at[pl.ds(b * G, G)], gsem[b]
            ).wait()

        def transpose(b):
            ridx = [iota + (b * G + 16 * k) for k in range(8)]

            @plsc.parallel_loop(0, D, unroll=4)
            def _(c):
                ct = c // 8
                cm = c % 8
                cidx = jnp.full((16,), c, jnp.int32)
                for k in range(8):
                    vals = plsc.load_gather(rows, [ridx[k], cidx])
                    tiles[b * CT + ct, pl.ds(cm * G + 16 * k, 16)] = vals

        def write(s, b):
            pltpu.async_copy(
                tiles.at[pl.ds(b * CT, CT)],
                out_hbm.at[s, :, wid, :],
                osem[b],
            )

        def wait_write(b):
            pltpu.make_async_copy(
                tiles.at[pl.ds(b * CT, CT)],
                out_hbm.at[0, :, 0, :],
                osem[b],
            ).wait()

        def retire(s, b):
            drain(b)
            transpose(b)
            write(s, b)

        fire(0, 0)
        fire(1, 1)
        fire(2, 2)
        retire(0, 0)
        fire(3, 3)
        retire(1, 1)

        def body(g, carry):
            s0 = NBUF + g * NBUF
            for b in range(NBUF):
                s = s0 + b
                wait_write(b)
                fire(s, b)
                retire(s - LA, (b + NBUF - LA) % NBUF)
            return carry

        lax.fori_loop(0, n_outer, body, 0)

        for s in range(SEQ - LA, SEQ):
            retire(s, s % NBUF)
        for b in range(NBUF):
            wait_write(b)

    return k


def kernel(indices, weight):
    BATCH, SEQ = indices.shape
    idx_t = jnp.transpose(indices.astype(jnp.int32))
    out4 = _make_kernel(BATCH, SEQ)(weight, idx_t)
    out5 = jnp.reshape(out4, (SEQ, CT, NW, 8, G))
    out = jnp.transpose(out5, (2, 4, 0, 1, 3))
    return jnp.reshape(out, (BATCH, SEQ, D))


# speedup vs baseline: 1.4971x; 1.0147x over previous
import functools

import jax
import jax.numpy as jnp
from jax import lax
from jax.experimental import pallas as pl
from jax.experimental.pallas import tpu as pltpu
from jax.experimental.pallas import tpu_sc as plsc

D = 64
NC, NS = 2, 16
NW = NC * NS
G = 128
NF = 2
CHUNK = NF * G
NBUF = 4
LA = 2


def _make_kernel(B):
    b_per_w = B // NW
    n_iter = b_per_w // CHUNK
    assert b_per_w % CHUNK == 0 and (n_iter - NBUF) % NBUF == 0
    n_outer = (n_iter - NBUF) // NBUF
    idx_rows = n_iter * NF
    mesh = plsc.VectorSubcoreMesh(core_axis_name="c", subcore_axis_name="s")

    @functools.partial(
        pl.kernel,
        mesh=mesh,
        out_type=jax.ShapeDtypeStruct((B, D), jnp.float32),
        scratch_types=[
            pltpu.VMEM((idx_rows, G), jnp.int32),
            pltpu.VMEM((NBUF * CHUNK, D), jnp.float32),
        ]
        + [pltpu.SemaphoreType.DMA] * (2 * NBUF),
        compiler_params=pltpu.CompilerParams(
            use_tc_tiling_on_sc=False, needs_layout_passes=False
        ),
    )
    def k(table, idx_hbm, out_hbm, idx_all, rows, *sems):
        gsem, osem = sems[:NBUF], sems[NBUF:]
        wid = lax.axis_index("s") * NC + lax.axis_index("c")
        pltpu.sync_copy(idx_hbm.at[pl.ds(wid * idx_rows, idx_rows)], idx_all)
        out_base = wid * b_per_w

        def fire(c, b):
            for j in range(NF):
                pltpu.async_copy(
                    table.at[idx_all.at[c * NF + j]],
                    rows.at[pl.ds(b * CHUNK + j * G, G)],
                    gsem[b],
                )

        def drain(b):
            pltpu.make_async_copy(
                table.at[pl.ds(0, CHUNK)],
                rows.at[pl.ds(b * CHUNK, CHUNK)],
                gsem[b],
            ).wait()

        def write(c, b):
            pltpu.async_copy(
                rows.at[pl.ds(b * CHUNK, CHUNK)],
                out_hbm.at[pl.ds(out_base + c * CHUNK, CHUNK)],
                osem[b],
            )

        def wait_write(b):
            pltpu.make_async_copy(
                rows.at[pl.ds(b * CHUNK, CHUNK)],
                out_hbm.at[pl.ds(0, CHUNK)],
                osem[b],
            ).wait()

        fire(0, 0)
        fire(1, 1)
        fire(2, 2)
        drain(0)
        write(0, 0)
        fire(3, 3)
        drain(1)
        write(1, 1)

        def body(g, carry):
            c0 = NBUF + g * NBUF
            for b in range(NBUF):
                c = c0 + b
                wait_write(b)
                fire(c, b)
                bd = (b + NBUF - LA) % NBUF
                drain(bd)
                write(c - LA, bd)
            return carry

        lax.fori_loop(0, n_outer, body, 0)

        for c in range(n_iter - LA, n_iter):
            b = c % NBUF
            drain(b)
            write(c, b)
        for b in range(NBUF):
            wait_write(b)

    return k


def kernel(indices, weight):
    B = indices.shape[0] * indices.shape[1]
    idx = jnp.reshape(indices.astype(jnp.int32), (B // G, G))
    out = _make_kernel(B)(weight, idx)
    return jnp.reshape(out, (*indices.shape, D))


# speedup vs baseline: 1.8279x; 1.2210x over previous
import functools

import jax
import jax.numpy as jnp
from jax import lax
from jax.experimental import pallas as pl
from jax.experimental.pallas import tpu as pltpu
from jax.experimental.pallas import tpu_sc as plsc

D = 64
DP = 2 * D
NC, NS = 2, 16
NW = NC * NS
G = 128
CHUNK = G
NBUF = 4
LA = 2


def _make_kernel(B):
    b_per_w = B // NW
    n_iter = b_per_w // CHUNK
    assert b_per_w % CHUNK == 0 and (n_iter - NBUF) % NBUF == 0
    n_outer = (n_iter - NBUF) // NBUF
    mesh = plsc.VectorSubcoreMesh(core_axis_name="c", subcore_axis_name="s")

    @functools.partial(
        pl.kernel,
        mesh=mesh,
        out_type=jax.ShapeDtypeStruct((B, DP), jnp.float32),
        scratch_types=[
            pltpu.VMEM((n_iter, G), jnp.int32),
            pltpu.VMEM((NBUF * CHUNK, DP), jnp.float32),
        ]
        + [pltpu.SemaphoreType.DMA] * (2 * NBUF),
        compiler_params=pltpu.CompilerParams(
            use_tc_tiling_on_sc=False, needs_layout_passes=False
        ),
    )
    def k(table, idx_hbm, out_hbm, idx_all, rows, *sems):
        gsem, osem = sems[:NBUF], sems[NBUF:]
        wid = lax.axis_index("s") * NC + lax.axis_index("c")
        pltpu.sync_copy(idx_hbm.at[pl.ds(wid * n_iter, n_iter)], idx_all)
        out_base = wid * b_per_w

        def fire(c, b):
            pltpu.async_copy(
                table.at[idx_all.at[c]],
                rows.at[pl.ds(b * CHUNK, CHUNK)],
                gsem[b],
            )

        def drain(b):
            pltpu.make_async_copy(
                table.at[pl.ds(0, CHUNK)],
                rows.at[pl.ds(b * CHUNK, CHUNK)],
                gsem[b],
            ).wait()

        def write(c, b):
            pltpu.async_copy(
                rows.at[pl.ds(b * CHUNK, CHUNK)],
                out_hbm.at[pl.ds(out_base + c * CHUNK, CHUNK)],
                osem[b],
            )

        def wait_write(b):
            pltpu.make_async_copy(
                rows.at[pl.ds(b * CHUNK, CHUNK)],
                out_hbm.at[pl.ds(0, CHUNK)],
                osem[b],
            ).wait()

        fire(0, 0)
        fire(1, 1)
        fire(2, 2)
        drain(0)
        write(0, 0)
        fire(3, 3)
        drain(1)
        write(1, 1)

        def body(g, carry):
            c0 = NBUF + g * NBUF
            for b in range(NBUF):
                c = c0 + b
                wait_write(b)
                fire(c, b)
                bd = (b + NBUF - LA) % NBUF
                drain(bd)
                write(c - LA, bd)
            return carry

        lax.fori_loop(0, n_outer, body, 0)

        for c in range(n_iter - LA, n_iter):
            b = c % NBUF
            drain(b)
            write(c, b)
        for b in range(NBUF):
            wait_write(b)

    return k


def kernel(indices, weight):
    B = indices.shape[0] * indices.shape[1]
    wt = jnp.pad(weight, ((0, 0), (0, D)))
    idx = jnp.reshape(indices.astype(jnp.int32), (B // G, G))
    out_p = _make_kernel(B)(wt, idx)
    out = out_p[:, :D]
    return jnp.reshape(out, (*indices.shape, D))


# speedup vs baseline: 2.1355x; 1.1683x over previous
import functools

import jax
import jax.numpy as jnp
from jax import lax
from jax.experimental import pallas as pl
from jax.experimental.pallas import tpu as pltpu
from jax.experimental.pallas import tpu_sc as plsc

D = 64
DP = 2 * D
NC, NS = 2, 16
NW = NC * NS
G = 128
CHUNK = G
NBUF = 4
LA = 2


def _make_kernel(B):
    b_per_w = B // NW
    n_iter = b_per_w // CHUNK
    assert b_per_w % CHUNK == 0 and (n_iter - NBUF) % NBUF == 0
    n_outer = (n_iter - NBUF) // NBUF
    mesh = plsc.VectorSubcoreMesh(core_axis_name="c", subcore_axis_name="s")

    @functools.partial(
        pl.kernel,
        mesh=mesh,
        out_type=jax.ShapeDtypeStruct((B, DP), jnp.float32),
        scratch_types=[
            pltpu.VMEM((n_iter, G), jnp.int32),
            pltpu.VMEM((NBUF * CHUNK, D), jnp.float32),
        ]
        + [pltpu.SemaphoreType.DMA] * (2 * NBUF),
        compiler_params=pltpu.CompilerParams(
            use_tc_tiling_on_sc=False, needs_layout_passes=False
        ),
    )
    def k(table, idx_hbm, out_hbm, idx_all, rows, *sems):
        gsem, osem = sems[:NBUF], sems[NBUF:]
        wid = lax.axis_index("s") * NC + lax.axis_index("c")
        pltpu.sync_copy(idx_hbm.at[pl.ds(wid * n_iter, n_iter)], idx_all)
        out_base = wid * b_per_w

        def fire(c, b):
            pltpu.async_copy(
                table.at[idx_all.at[c]],
                rows.at[pl.ds(b * CHUNK, CHUNK)],
                gsem[b],
            )

        def drain(b):
            pltpu.make_async_copy(
                table.at[pl.ds(0, CHUNK)],
                rows.at[pl.ds(b * CHUNK, CHUNK)],
                gsem[b],
            ).wait()

        def write(c, b):
            pltpu.async_copy(
                rows.at[pl.ds(b * CHUNK, CHUNK)],
                out_hbm.at[pl.ds(out_base + c * CHUNK, CHUNK), pl.ds(0, D)],
                osem[b],
            )

        def wait_write(b):
            pltpu.make_async_copy(
                rows.at[pl.ds(b * CHUNK, CHUNK)],
                out_hbm.at[pl.ds(0, CHUNK), pl.ds(0, D)],
                osem[b],
            ).wait()

        fire(0, 0)
        fire(1, 1)
        fire(2, 2)
        drain(0)
        write(0, 0)
        fire(3, 3)
        drain(1)
        write(1, 1)

        def body(g, carry):
            c0 = NBUF + g * NBUF
            for b in range(NBUF):
                c = c0 + b
                wait_write(b)
                fire(c, b)
                bd = (b + NBUF - LA) % NBUF
                drain(bd)
                write(c - LA, bd)
            return carry

        lax.fori_loop(0, n_outer, body, 0)

        for c in range(n_iter - LA, n_iter):
            b = c % NBUF
            drain(b)
            write(c, b)
        for b in range(NBUF):
            wait_write(b)

    return k


def kernel(indices, weight):
    B = indices.shape[0] * indices.shape[1]
    V = weight.shape[0]
    wt = jnp.reshape(jnp.pad(weight, ((0, 0), (0, D))), (2 * V, D))
    idx = jnp.reshape(indices.astype(jnp.int32) * 2, (B // G, G))
    out_p = _make_kernel(B)(wt, idx)
    out = out_p[:, :D]
    return jnp.reshape(out, (*indices.shape, D))


# speedup vs baseline: 2.1370x; 1.0007x over previous
import functools

import jax
import jax.numpy as jnp
from jax import lax
from jax.experimental import pallas as pl
from jax.experimental.pallas import tpu as pltpu
from jax.experimental.pallas import tpu_sc as plsc

D = 64
DP = 2 * D
NC, NS = 2, 16
NW = NC * NS
G = 128
CHUNK = G
NBUF = 8
LA = 4


def _make_kernel(B):
    b_per_w = B // NW
    n_iter = b_per_w // CHUNK
    assert b_per_w % CHUNK == 0 and (n_iter - NBUF) % NBUF == 0
    n_outer = (n_iter - NBUF) // NBUF
    mesh = plsc.VectorSubcoreMesh(core_axis_name="c", subcore_axis_name="s")

    @functools.partial(
        pl.kernel,
        mesh=mesh,
        out_type=jax.ShapeDtypeStruct((B, DP), jnp.float32),
        scratch_types=[
            pltpu.VMEM((n_iter, G), jnp.int32),
            pltpu.VMEM((NBUF * CHUNK, D), jnp.float32),
        ]
        + [pltpu.SemaphoreType.DMA] * (2 * NBUF),
        compiler_params=pltpu.CompilerParams(
            use_tc_tiling_on_sc=False, needs_layout_passes=False
        ),
    )
    def k(table, idx_hbm, out_hbm, idx_all, rows, *sems):
        gsem, osem = sems[:NBUF], sems[NBUF:]
        wid = lax.axis_index("s") * NC + lax.axis_index("c")
        pltpu.sync_copy(idx_hbm.at[pl.ds(wid * n_iter, n_iter)], idx_all)
        out_base = wid * b_per_w

        def fire(c, b):
            pltpu.async_copy(
                table.at[idx_all.at[c]],
                rows.at[pl.ds(b * CHUNK, CHUNK)],
                gsem[b],
            )

        def drain(b):
            pltpu.make_async_copy(
                table.at[pl.ds(0, CHUNK)],
                rows.at[pl.ds(b * CHUNK, CHUNK)],
                gsem[b],
            ).wait()

        def write(c, b):
            pltpu.async_copy(
                rows.at[pl.ds(b * CHUNK, CHUNK)],
                out_hbm.at[pl.ds(out_base + c * CHUNK, CHUNK), pl.ds(0, D)],
                osem[b],
            )

        def wait_write(b):
            pltpu.make_async_copy(
                rows.at[pl.ds(b * CHUNK, CHUNK)],
                out_hbm.at[pl.ds(0, CHUNK), pl.ds(0, D)],
                osem[b],
            ).wait()

        for c in range(LA):
            fire(c, c)
        for c in range(LA, NBUF):
            fire(c, c)
            drain(c - LA)
            write(c - LA, c - LA)

        def body(g, carry):
            c0 = NBUF + g * NBUF
            for b in range(NBUF):
                c = c0 + b
                wait_write(b)
                fire(c, b)
                bd = (b + NBUF - LA) % NBUF
                drain(bd)
                write(c - LA, bd)
            return carry

        lax.fori_loop(0, n_outer, body, 0)

        for c in range(n_iter - LA, n_iter):
            b = c % NBUF
            drain(b)
            write(c, b)
        for b in range(NBUF):
            wait_write(b)

    return k


def kernel(indices, weight):
    B = indices.shape[0] * indices.shape[1]
    V = weight.shape[0]
    wt = jnp.reshape(jnp.pad(weight, ((0, 0), (0, D))), (2 * V, D))
    idx = jnp.reshape(indices.astype(jnp.int32) * 2, (B // G, G))
    out_p = _make_kernel(B)(wt, idx)
    out = out_p[:, :D]
    return jnp.reshape(out, (*indices.shape, D))
